# baseline (speedup 1.0000x reference)
import functools

import numpy as np
import jax
import jax.numpy as jnp
from jax import lax
from jax.experimental import pallas as pl
from jax.experimental.pallas import tpu as pltpu
from jax.experimental.pallas import tpu_sc as plsc

N_COLS = 100000
N_S = 4096
N_CH = 3
NW = 32
S_PER_W = N_S // NW


def _build_tables():
    rng = np.random.default_rng(0)
    a_idx = rng.permutation(N_COLS)[:N_S].astype(np.int64)
    b_idx = rng.permutation(N_COLS)[:N_S].astype(np.int64)

    q = np.arange(S_PER_W, dtype=np.int64)
    s, rloc = q // 16, q % 16
    p_a = rloc * 8 + s
    gidx_a = np.empty((NW * N_CH, S_PER_W), np.int32)
    gidx_b = np.empty((NW * N_CH, S_PER_W), np.int32)
    for w in range(NW):
        for c in range(N_CH):
            gidx_a[w * N_CH + c] = c * N_COLS + a_idx[w * S_PER_W + p_a]
            gidx_b[w * N_CH + c] = 6 * N_COLS + c * N_COLS + b_idx[w * S_PER_W + q]
    return gidx_a, gidx_b


_GIDX_A, _GIDX_B = _build_tables()


def _sc_gather_body(ab_hbm, gidx_a, gidx_b, out_a, out_b,
                    idx_v, vals_v, prep_v, gsem, osem):
    wid = lax.axis_index("s") * 2 + lax.axis_index("c")

    pltpu.sync_copy(gidx_a.at[pl.ds(wid * N_CH, N_CH)], idx_v.at[0])
    pltpu.sync_copy(gidx_b.at[pl.ds(wid * N_CH, N_CH)], idx_v.at[1])
    gathers = []
    for side in range(2):
        for c in range(N_CH):
            gathers.append(pltpu.async_copy(
                ab_hbm.at[idx_v.at[side, c]], vals_v.at[side, c], gsem))
    for g in gathers:
        g.wait()

    outs = []
    for c in range(N_CH):
        for s in range(8):
            outs.append(pltpu.async_copy(
                vals_v.at[0, c, pl.ds(s * 16, 16)],
                out_a.at[c, s, pl.ds(wid * 16, 16)],
                osem))

    for k in range(S_PER_W // 16):
        v0 = vals_v[1, 0, pl.ds(k * 16, 16)]
        v1 = vals_v[1, 1, pl.ds(k * 16, 16)]
        v2 = vals_v[1, 2, pl.ds(k * 16, 16)]
        prep_v[0, pl.ds(k * 16, 16)] = -2.0 * v0
        prep_v[1, pl.ds(k * 16, 16)] = -2.0 * v1
        prep_v[2, pl.ds(k * 16, 16)] = -2.0 * v2
        prep_v[3, pl.ds(k * 16, 16)] = v0 * v0 + v1 * v1 + v2 * v2
    for c in range(4):
        outs.append(pltpu.async_copy(
            prep_v.at[c], out_b.at[c, pl.ds(wid * S_PER_W, S_PER_W)], osem))
    for o in outs:
        o.wait()


@functools.cache
def _sc_gather():
    return pl.kernel(
        _sc_gather_body,
        mesh=plsc.VectorSubcoreMesh(core_axis_name="c", subcore_axis_name="s"),
        compiler_params=pltpu.CompilerParams(
            needs_layout_passes=False, use_tc_tiling_on_sc=False),
        out_type=[
            jax.ShapeDtypeStruct((N_CH, 8, N_S // 8), jnp.float32),
            jax.ShapeDtypeStruct((4, N_S), jnp.float32),
        ],
        scratch_types=[
            pltpu.VMEM((2, N_CH, S_PER_W), jnp.int32),
            pltpu.VMEM((2, N_CH, S_PER_W), jnp.float32),
            pltpu.VMEM((4, S_PER_W), jnp.float32),
            pltpu.SemaphoreType.DMA,
            pltpu.SemaphoreType.DMA,
        ],
    )


A_BLK = 32
N_ABLK = N_S // A_BLK
N_BBLK = N_S // 128


def _tc_chamfer_body(a_ref, b_ref, out_ref, bb_ref, ab_ref, mb_ref):
    for c in range(4):
        bb_ref[c] = jnp.broadcast_to(b_ref[c:c + 1, :], (8, N_S))

    def abuild(k2, carry):
        for half in range(2):
            k = k2 * 2 + half
            base = pl.multiple_of((k // 8) * 128, 128)
            sh = (k % 8) * 16
            for c in range(N_CH):
                av = pltpu.roll(a_ref[c, :, pl.ds(base, 128)], -sh, 1)
                for u in range(16):
                    ab_ref[c, k * 16 + u] = jnp.broadcast_to(
                        av[:, u:u + 1], (8, 128))
        return carry

    lax.fori_loop(0, N_S // 8 // 32, abuild, 0)

    def blk_step(blk, carry):
        grp = []
        accs = []
        for g in range(4):
            rr = blk * 4 + g
            grp.append((ab_ref[0, rr], ab_ref[1, rr], ab_ref[2, rr]))
            accs.append(jnp.full((8, 128), jnp.inf, dtype=jnp.float32))
        def bb_step(bb, accs):
            off = pl.multiple_of(bb * 128, 128)
            b0 = bb_ref[0, :, pl.ds(off, 128)]
            b1 = bb_ref[1, :, pl.ds(off, 128)]
            b2 = bb_ref[2, :, pl.ds(off, 128)]
            nb = bb_ref[3, :, pl.ds(off, 128)]
            out = []
            for g in range(4):
                a0, a1, a2 = grp[g]
                v = nb + b0 * a0 + b1 * a1 + b2 * a2
                out.append(jnp.minimum(accs[g], v))
            return tuple(out)

        accs = lax.fori_loop(0, N_BBLK, bb_step, tuple(accs), unroll=4)
        for g in range(4):
            a0, a1, a2 = grp[g]
            na = a0 * a0 + a1 * a1 + a2 * a2
            mb_ref[blk * 4 + g] = accs[g] + na
        return carry

    lax.fori_loop(0, N_ABLK, blk_step, 0)

    def fin_step(i, sacc):
        return sacc + jnp.min(mb_ref[i], axis=1, keepdims=True)

    sacc = lax.fori_loop(0, N_S // 8, fin_step,
                         jnp.zeros((8, 1), dtype=jnp.float32), unroll=16)
    out_ref[0, 0] = jnp.sum(sacc)


_tc_chamfer = pl.pallas_call(
    _tc_chamfer_body,
    out_shape=jax.ShapeDtypeStruct((1, 1), jnp.float32),
    in_specs=[
        pl.BlockSpec(memory_space=pltpu.VMEM),
        pl.BlockSpec(memory_space=pltpu.VMEM),
    ],
    out_specs=pl.BlockSpec(memory_space=pltpu.SMEM),
    scratch_shapes=[
        pltpu.VMEM((4, 8, N_S), jnp.float32),
        pltpu.VMEM((N_CH, N_S // 8, 8, 128), jnp.float32),
        pltpu.VMEM((N_S // 8, 8, 128), jnp.float32),
    ],
)


@jax.jit
def kernel(a, b):
    ab_flat = jnp.concatenate([a.reshape(6 * N_COLS), b.reshape(6 * N_COLS)])
    a_g, b_g = _sc_gather()(
        ab_flat, jnp.asarray(_GIDX_A), jnp.asarray(_GIDX_B))
    return _tc_chamfer(a_g, b_g)[0, 0]
